# baseline (speedup 1.0000x reference)
import functools

import jax
import jax.numpy as jnp
from jax import lax
from jax.experimental import pallas as pl
from jax.experimental.pallas import tpu as pltpu
from jax.experimental.pallas import tpu_sc as plsc

NPTS = 360
NGT = 360
NP_PAD = 368
NG_PAD = 384
N_CHUNKS = NP_PAD // 16
NLVL = 3
NPOLY = 32
WEIGHTS = (0.2, 0.3, 0.5)

_mesh = plsc.VectorSubcoreMesh(core_axis_name="c", subcore_axis_name="s")


@functools.partial(
    pl.kernel,
    mesh=_mesh,
    out_type=jax.ShapeDtypeStruct((NLVL, NPOLY, NP_PAD), jnp.float32),
    scratch_types=[
        pltpu.VMEM((NP_PAD,), jnp.float32),
        pltpu.VMEM((NP_PAD,), jnp.float32),
        pltpu.VMEM((NG_PAD,), jnp.float32),
        pltpu.VMEM((NG_PAD,), jnp.float32),
        pltpu.VMEM((NP_PAD,), jnp.float32),
    ],
)
def _sc_min_d2(predx, predy, gtx, gty, out, px_v, py_v, gx_v, gy_v, min_v):
    wid = lax.axis_index("s") * 2 + lax.axis_index("c")
    pltpu.sync_copy(gtx.at[wid], gx_v)
    pltpu.sync_copy(gty.at[wid], gy_v)
    for n in range(NLVL):
        pltpu.sync_copy(predx.at[n, wid], px_v)
        pltpu.sync_copy(predy.at[n, wid], py_v)
        for c in range(N_CHUNKS):
            px = px_v[pl.ds(c * 16, 16)]
            py = py_v[pl.ds(c * 16, 16)]

            def body(j, acc, px=px, py=py):
                gxj = gx_v[pl.ds(j, 16)][0]
                gyj = gy_v[pl.ds(j, 16)][0]
                dx = px - gxj
                dy = py - gyj
                return jnp.minimum(acc, dx * dx + dy * dy)

            acc0 = jnp.full((16,), jnp.inf, dtype=jnp.float32)
            acc = lax.fori_loop(0, NGT, body, acc0)
            min_v[pl.ds(c * 16, 16)] = acc
        pltpu.sync_copy(min_v, out.at[n, wid])


def _tc_reduce_body(d2_ref, o_ref):
    d2 = d2_ref[...]
    d = jnp.sqrt(jnp.maximum(d2, 0.0))
    row = lax.broadcasted_iota(jnp.int32, d.shape, 0)
    lane = lax.broadcasted_iota(jnp.int32, d.shape, 1)
    scale = 1.0 / (NPTS * NPOLY * NLVL)
    w = jnp.where(row < NPOLY, WEIGHTS[0] * scale,
                  jnp.where(row < 2 * NPOLY, WEIGHTS[1] * scale,
                            WEIGHTS[2] * scale))
    val = jnp.where(lane < NPTS, d * w, 0.0)
    o_ref[0, 0] = jnp.sum(val)


_tc_reduce = pl.pallas_call(
    _tc_reduce_body,
    out_shape=jax.ShapeDtypeStruct((1, 1), jnp.float32),
    out_specs=pl.BlockSpec(memory_space=pltpu.SMEM),
)


def kernel(pred, gt):
    pts = pred[:, 0, :, :, 1:]
    predx = jnp.pad(pts[..., 0], ((0, 0), (0, 0), (0, NP_PAD - NPTS)))
    predy = jnp.pad(pts[..., 1], ((0, 0), (0, 0), (0, NP_PAD - NPTS)))
    gtx = jnp.pad(gt[0, :, :, 0], ((0, 0), (0, NG_PAD - NGT)))
    gty = jnp.pad(gt[0, :, :, 1], ((0, 0), (0, NG_PAD - NGT)))
    d2 = _sc_min_d2(predx, predy, gtx, gty)
    loss = _tc_reduce(d2.reshape(NLVL * NPOLY, NP_PAD))
    return loss[0, 0]


# speedup vs baseline: 1.8849x; 1.8849x over previous
import functools

import jax
import jax.numpy as jnp
from jax import lax
from jax.experimental import pallas as pl
from jax.experimental.pallas import tpu as pltpu
from jax.experimental.pallas import tpu_sc as plsc

NPTS = 360
NGT = 360
NP_PAD = 368
N_CHUNKS = NP_PAD // 16
NLVL = 3
NPOLY = 32
WEIGHTS = (0.2, 0.3, 0.5)
GROUP = 4

_mesh = plsc.VectorSubcoreMesh(core_axis_name="c", subcore_axis_name="s")


@functools.partial(
    pl.kernel,
    mesh=_mesh,
    out_type=jax.ShapeDtypeStruct((NLVL, NPOLY, NP_PAD), jnp.float32),
    scratch_types=[
        pltpu.VMEM((NP_PAD,), jnp.float32),
        pltpu.VMEM((NP_PAD,), jnp.float32),
        pltpu.VMEM((NGT, 16), jnp.float32),
        pltpu.VMEM((NGT, 16), jnp.float32),
        pltpu.VMEM((NP_PAD,), jnp.float32),
    ],
)
def _sc_min_d2(predx, predy, gtx, gty, out, px_v, py_v, gx_v, gy_v, min_v):
    wid = lax.axis_index("s") * 2 + lax.axis_index("c")
    pltpu.sync_copy(gtx.at[wid], gx_v)
    pltpu.sync_copy(gty.at[wid], gy_v)
    for n in range(NLVL):
        pltpu.sync_copy(predx.at[n, wid], px_v)
        pltpu.sync_copy(predy.at[n, wid], py_v)
        for g in range(0, N_CHUNKS, GROUP):
            k = min(GROUP, N_CHUNKS - g)
            pxs = [px_v[pl.ds((g + i) * 16, 16)] for i in range(k)]
            pys = [py_v[pl.ds((g + i) * 16, 16)] for i in range(k)]

            def body(j, accs, pxs=pxs, pys=pys, k=k):
                gxj = gx_v[j]
                gyj = gy_v[j]
                new = []
                for i in range(k):
                    dx = pxs[i] - gxj
                    dy = pys[i] - gyj
                    new.append(jnp.minimum(accs[i], dx * dx + dy * dy))
                return tuple(new)

            acc0 = tuple(
                jnp.full((16,), jnp.inf, dtype=jnp.float32) for _ in range(k))
            accs = lax.fori_loop(0, NGT, body, acc0, unroll=4)
            for i in range(k):
                min_v[pl.ds((g + i) * 16, 16)] = accs[i]
        pltpu.sync_copy(min_v, out.at[n, wid])


def _tc_reduce_body(d2_ref, o_ref):
    d2 = d2_ref[...]
    d = jnp.sqrt(jnp.maximum(d2, 0.0))
    row = lax.broadcasted_iota(jnp.int32, d.shape, 0)
    lane = lax.broadcasted_iota(jnp.int32, d.shape, 1)
    scale = 1.0 / (NPTS * NPOLY * NLVL)
    w = jnp.where(row < NPOLY, WEIGHTS[0] * scale,
                  jnp.where(row < 2 * NPOLY, WEIGHTS[1] * scale,
                            WEIGHTS[2] * scale))
    val = jnp.where(lane < NPTS, d * w, 0.0)
    o_ref[0, 0] = jnp.sum(val)


_tc_reduce = pl.pallas_call(
    _tc_reduce_body,
    out_shape=jax.ShapeDtypeStruct((1, 1), jnp.float32),
    out_specs=pl.BlockSpec(memory_space=pltpu.SMEM),
)


def kernel(pred, gt):
    pts = pred[:, 0, :, :, 1:]
    predx = jnp.pad(pts[..., 0], ((0, 0), (0, 0), (0, NP_PAD - NPTS)))
    predy = jnp.pad(pts[..., 1], ((0, 0), (0, 0), (0, NP_PAD - NPTS)))
    gtx = jnp.broadcast_to(gt[0, :, :, 0][..., None], (NPOLY, NGT, 16))
    gty = jnp.broadcast_to(gt[0, :, :, 1][..., None], (NPOLY, NGT, 16))
    d2 = _sc_min_d2(predx, predy, gtx, gty)
    loss = _tc_reduce(d2.reshape(NLVL * NPOLY, NP_PAD))
    return loss[0, 0]


# speedup vs baseline: 1.9658x; 1.0430x over previous
import functools

import jax
import jax.numpy as jnp
from jax import lax
from jax.experimental import pallas as pl
from jax.experimental.pallas import tpu as pltpu
from jax.experimental.pallas import tpu_sc as plsc

NPTS = 360
NGT = 360
NP_PAD = 384
N_CHUNKS = NP_PAD // 16
NLVL = 3
NPOLY = 32
WEIGHTS = (0.2, 0.3, 0.5)
GROUP = 6

_mesh = plsc.VectorSubcoreMesh(core_axis_name="c", subcore_axis_name="s")


@functools.partial(
    pl.kernel,
    mesh=_mesh,
    out_type=jax.ShapeDtypeStruct((NLVL, NPOLY, NP_PAD), jnp.float32),
    scratch_types=[
        pltpu.VMEM((NP_PAD,), jnp.float32),
        pltpu.VMEM((NP_PAD,), jnp.float32),
        pltpu.VMEM((NGT * 16,), jnp.float32),
        pltpu.VMEM((NGT * 16,), jnp.float32),
        pltpu.VMEM((NGT * 16,), jnp.float32),
        pltpu.VMEM((NGT * 16,), jnp.float32),
        pltpu.VMEM((NGT * 16,), jnp.float32),
        pltpu.VMEM((NP_PAD,), jnp.float32),
    ],
)
def _sc_min_d2(predx, predy, gtx, gty, out,
               px_v, py_v, gx_v, gy_v, ax_v, ay_v, c_v, min_v):
    wid = lax.axis_index("s") * 2 + lax.axis_index("c")
    pltpu.sync_copy(gtx.at[wid], gx_v)
    pltpu.sync_copy(gty.at[wid], gy_v)

    def prep(j, _):
        gx = gx_v[pl.ds(j * 16, 16)]
        gy = gy_v[pl.ds(j * 16, 16)]
        ax_v[pl.ds(j * 16, 16)] = gx * (-2.0)
        ay_v[pl.ds(j * 16, 16)] = gy * (-2.0)
        c_v[pl.ds(j * 16, 16)] = gx * gx + gy * gy
        return 0

    lax.fori_loop(0, NGT, prep, 0, unroll=4)

    for n in range(NLVL):
        pltpu.sync_copy(predx.at[n, wid], px_v)
        pltpu.sync_copy(predy.at[n, wid], py_v)
        for g in range(0, N_CHUNKS, GROUP):
            k = min(GROUP, N_CHUNKS - g)
            pxs = [px_v[pl.ds((g + i) * 16, 16)] for i in range(k)]
            pys = [py_v[pl.ds((g + i) * 16, 16)] for i in range(k)]

            def body(j, accs, pxs=pxs, pys=pys, k=k):
                axj = ax_v[pl.ds(j * 16, 16)]
                ayj = ay_v[pl.ds(j * 16, 16)]
                cj = c_v[pl.ds(j * 16, 16)]
                new = []
                for i in range(k):
                    t = cj + axj * pxs[i] + ayj * pys[i]
                    new.append(jnp.minimum(accs[i], t))
                return tuple(new)

            acc0 = tuple(
                jnp.full((16,), jnp.inf, dtype=jnp.float32) for _ in range(k))
            accs = lax.fori_loop(0, NGT, body, acc0, unroll=6)
            for i in range(k):
                p2 = pxs[i] * pxs[i] + pys[i] * pys[i]
                min_v[pl.ds((g + i) * 16, 16)] = accs[i] + p2
        pltpu.sync_copy(min_v, out.at[n, wid])


def _tc_reduce_body(d2_ref, o_ref):
    d2 = d2_ref[...]
    d = jnp.sqrt(jnp.maximum(d2, 0.0))
    row = lax.broadcasted_iota(jnp.int32, d.shape, 0)
    lane = lax.broadcasted_iota(jnp.int32, d.shape, 1)
    scale = 1.0 / (NPTS * NPOLY * NLVL)
    w = jnp.where(row < NPOLY, WEIGHTS[0] * scale,
                  jnp.where(row < 2 * NPOLY, WEIGHTS[1] * scale,
                            WEIGHTS[2] * scale))
    val = jnp.where(lane < NPTS, d * w, 0.0)
    o_ref[0, 0] = jnp.sum(val)


_tc_reduce = pl.pallas_call(
    _tc_reduce_body,
    out_shape=jax.ShapeDtypeStruct((1, 1), jnp.float32),
    out_specs=pl.BlockSpec(memory_space=pltpu.SMEM),
)


def kernel(pred, gt):
    pts = pred[:, 0, :, :, 1:]
    predx = jnp.pad(pts[..., 0], ((0, 0), (0, 0), (0, NP_PAD - NPTS)))
    predy = jnp.pad(pts[..., 1], ((0, 0), (0, 0), (0, NP_PAD - NPTS)))
    gtx = jnp.broadcast_to(
        gt[0, :, :, 0][..., None], (NPOLY, NGT, 16)).reshape(NPOLY, NGT * 16)
    gty = jnp.broadcast_to(
        gt[0, :, :, 1][..., None], (NPOLY, NGT, 16)).reshape(NPOLY, NGT * 16)
    d2 = _sc_min_d2(predx, predy, gtx, gty)
    loss = _tc_reduce(d2.reshape(NLVL * NPOLY, NP_PAD))
    return loss[0, 0]
